# Initial kernel scaffold; baseline (speedup 1.0000x reference)
import jax
import jax.numpy as jnp
from jax.experimental import pallas as pl


def kernel(n_mem, n_last_update, c_mem, c_last_update, idx, t, n_idx, c_idx, memb_vals, time_w, time_b, n_proj_w, n_proj_b, c_proj_w, c_proj_b, out_w, out_b):
    raise NotImplementedError("write your pallas kernel here")


# speedup vs baseline: 9.2108x; 9.2108x over previous
import functools

import jax
import jax.numpy as jnp
from jax import lax
from jax.experimental import pallas as pl
from jax.experimental.pallas import tpu as pltpu
from jax.experimental.pallas import tpu_sc as plsc

N_QUERY = 10000
MEM_DIM = 128
TIME_DIM = 32
TW = 192
TPAD = 256
NW = 32
QPW = 320
NQ_PAD = NW * QPW
K = 128
NODE_CHUNK = 64


def _table_body(c_mem_ref, c_lu_ref, cw_ref, ow_ref, tw_ref, t_ref):
    w1u2 = jnp.dot(cw_ref[:MEM_DIM, :], ow_ref[MEM_DIM:, :],
                   preferred_element_type=jnp.float32)
    t_ref[:, :MEM_DIM] = jnp.dot(c_mem_ref[...], w1u2,
                                 preferred_element_type=jnp.float32)
    ph = c_lu_ref[...] * tw_ref[0, :]
    t_ref[:, MEM_DIM:MEM_DIM + TIME_DIM] = jnp.cos(ph)
    t_ref[:, MEM_DIM + TIME_DIM:TW] = jnp.sin(ph)
    t_ref[:, TW:] = jnp.zeros_like(t_ref[:, TW:])


def _epi_body(g_ref, nm_ref, nlu_ref, t_ref, tw_ref, tb_ref, npw_ref, npb_ref,
              cw_ref, cpb_ref, ow_ref, ob_ref, out_ref):
    u1 = ow_ref[:MEM_DIM, :]
    u2 = ow_ref[MEM_DIM:, :]
    wn1u1 = jnp.dot(npw_ref[:MEM_DIM, :], u1, preferred_element_type=jnp.float32)
    wn2u1 = jnp.dot(npw_ref[MEM_DIM:, :], u1, preferred_element_type=jnp.float32)
    w2u2 = jnp.dot(cw_ref[MEM_DIM:, :], u2, preferred_element_type=jnp.float32)
    t = t_ref[...]
    ph_t = t * tw_ref[0, :] + tb_ref[...]
    a = jnp.cos(ph_t)
    b = jnp.sin(ph_t)
    ntime = jnp.cos((t - nlu_ref[...]) * tw_ref[0, :] + tb_ref[...])
    g2 = g_ref[:, MEM_DIM:MEM_DIM + TIME_DIM]
    g3 = g_ref[:, MEM_DIM + TIME_DIM:TW]
    out = jnp.dot(nm_ref[...], wn1u1, preferred_element_type=jnp.float32)
    out += jnp.dot(ntime, wn2u1, preferred_element_type=jnp.float32)
    out += g_ref[:, :MEM_DIM]
    out += jnp.dot(a * g2 + b * g3, w2u2, preferred_element_type=jnp.float32)
    out += g_ref[:, TW:TW + 1] * jnp.dot(cpb_ref[...], u2,
                                         preferred_element_type=jnp.float32)
    out += jnp.dot(npb_ref[...], u1, preferred_element_type=jnp.float32)
    out += ob_ref[...]
    out_ref[...] = out


def _sc_body(t_hbm, cidx_hbm, nidx_hbm, v_hbm, idx_hbm, nmem_hbm, nlu_hbm,
             bounds_hbm, g_hbm, nmemg_hbm, nlug_hbm,
             staging, rows, cidx_b, nidx_b, v_b, nrows_b, nlub_b, qidx_b,
             ridx_b, bounds_b, sem):
    wid = lax.axis_index("s") * 2 + lax.axis_index("c")
    qbase = wid * QPW
    zeros16 = jnp.zeros((16,), jnp.float32)

    pltpu.sync_copy(bounds_hbm, bounds_b)
    lo = plsc.load_gather(bounds_b, [jnp.full((16,), wid, jnp.int32)])[0]
    hi = plsc.load_gather(bounds_b, [jnp.full((16,), wid + 1, jnp.int32)])[0]

    def _zero_row(r, _):
        for j in range(TPAD // 16):
            staging[r, pl.ds(16 * j, 16)] = zeros16
        return 0
    lax.fori_loop(0, QPW, _zero_row, 0)

    def _flush(cur_q, accs, acc_s):
        ql = jnp.maximum(cur_q - qbase, 0)
        for j in range(TW // 16):
            staging[ql, pl.ds(16 * j, 16)] = accs[j]
        staging[ql, pl.ds(TW, 16)] = acc_s

    def _chunk(c, carry):
        start = pl.multiple_of(c * K, K)
        pltpu.sync_copy(cidx_hbm.at[pl.ds(start, K)], cidx_b)
        pltpu.sync_copy(nidx_hbm.at[pl.ds(start, K)], nidx_b)
        pltpu.sync_copy(v_hbm.at[pl.ds(start, K)], v_b)
        pltpu.async_copy(t_hbm.at[cidx_b], rows, sem).wait()
        kstart = jnp.maximum(lo - start, 0)
        kend = jnp.minimum(hi - start, K)

        def _memb(k, carry):
            cur_q = carry[0]
            accs = list(carry[1:1 + TW // 16])
            acc_s = carry[1 + TW // 16]
            kvec = jnp.full((16,), k, jnp.int32)
            q = plsc.load_gather(nidx_b, [kvec])[0]
            changed = q != cur_q

            @pl.when(changed)
            def _():
                _flush(cur_q, accs, acc_s)

            accs = [jnp.where(changed, zeros16, a) for a in accs]
            acc_s = jnp.where(changed, zeros16, acc_s)
            vsplat = plsc.load_gather(v_b, [kvec])
            for j in range(TW // 16):
                accs[j] = accs[j] + vsplat * rows[k, pl.ds(16 * j, 16)]
            acc_s = acc_s + vsplat
            return (q, *accs, acc_s)

        return lax.fori_loop(kstart, kend, _memb, carry)

    carry0 = (jnp.int32(-1),) + tuple(zeros16 for _ in range(TW // 16 + 1))
    c0 = lo // K
    c1 = (hi + K - 1) // K
    carry = lax.fori_loop(c0, c1, _chunk, carry0)
    cur_q = carry[0]

    @pl.when(cur_q >= 0)
    def _():
        _flush(cur_q, list(carry[1:1 + TW // 16]), carry[1 + TW // 16])

    pltpu.sync_copy(staging, g_hbm.at[pl.ds(qbase, QPW)])

    for nc in range(QPW // NODE_CHUNK):
        nb = qbase + NODE_CHUNK * nc
        pltpu.sync_copy(idx_hbm.at[pl.ds(nb, NODE_CHUNK)], qidx_b)
        pltpu.async_copy(nmem_hbm.at[qidx_b], nrows_b, sem).wait()
        pltpu.sync_copy(nrows_b, nmemg_hbm.at[pl.ds(nb, NODE_CHUNK)])
        for grp in range(NODE_CHUNK // 16):
            qv = qidx_b[pl.ds(16 * grp, 16)]
            ridx_b[pl.ds(16 * grp, 16)] = lax.shift_right_logical(
                qv, jnp.full((16,), 7, jnp.int32))
        pltpu.async_copy(nlu_hbm.at[ridx_b], nrows_b, sem).wait()
        for grp in range(NODE_CHUNK // 16):
            qv = qidx_b[pl.ds(16 * grp, 16)]
            col = lax.bitwise_and(qv, jnp.full((16,), 127, jnp.int32))
            row = lax.broadcasted_iota(jnp.int32, (16,), 0) + 16 * grp
            nlub_b[pl.ds(16 * grp, 16)] = plsc.load_gather(
                nrows_b, [row, col])
        pltpu.sync_copy(nlub_b, nlug_hbm.at[pl.ds(nb, NODE_CHUNK)])


_sc_call = functools.partial(
    pl.kernel,
    out_type=(
        jax.ShapeDtypeStruct((NQ_PAD, TPAD), jnp.float32),
        jax.ShapeDtypeStruct((NQ_PAD, MEM_DIM), jnp.float32),
        jax.ShapeDtypeStruct((NQ_PAD,), jnp.float32),
    ),
    mesh=plsc.VectorSubcoreMesh(core_axis_name="c", subcore_axis_name="s"),
    compiler_params=pltpu.CompilerParams(needs_layout_passes=False),
    scratch_types=[
        pltpu.VMEM((QPW, TPAD), jnp.float32),
        pltpu.VMEM((K, TPAD), jnp.float32),
        pltpu.VMEM((K,), jnp.int32),
        pltpu.VMEM((K,), jnp.int32),
        pltpu.VMEM((K,), jnp.float32),
        pltpu.VMEM((NODE_CHUNK, MEM_DIM), jnp.float32),
        pltpu.VMEM((NODE_CHUNK,), jnp.float32),
        pltpu.VMEM((NODE_CHUNK,), jnp.int32),
        pltpu.VMEM((NODE_CHUNK,), jnp.int32),
        pltpu.VMEM((128,), jnp.int32),
        pltpu.SemaphoreType.DMA,
    ],
)(_sc_body)


def kernel(n_mem, n_last_update, c_mem, c_last_update, idx, t, n_idx, c_idx,
           memb_vals, time_w, time_b, n_proj_w, n_proj_b, c_proj_w, c_proj_b,
           out_w, out_b):
    n_comm = c_mem.shape[0]
    n_nodes = n_mem.shape[0]
    f32 = jnp.float32

    tb = n_comm // 5
    table = pl.pallas_call(
        _table_body,
        grid=(5,),
        in_specs=[
            pl.BlockSpec((tb, MEM_DIM), lambda i: (i, 0)),
            pl.BlockSpec((tb, 1), lambda i: (i, 0)),
            pl.BlockSpec((160, MEM_DIM), lambda i: (0, 0)),
            pl.BlockSpec((2 * MEM_DIM, MEM_DIM), lambda i: (0, 0)),
            pl.BlockSpec((1, TIME_DIM), lambda i: (0, 0)),
        ],
        out_specs=pl.BlockSpec((tb, TPAD), lambda i: (i, 0)),
        out_shape=jax.ShapeDtypeStruct((n_comm, TPAD), f32),
    )(c_mem, c_last_update.reshape(n_comm, 1).astype(f32), c_proj_w, out_w,
      time_w)

    idx_i = idx.astype(jnp.int32)
    cidx_i = c_idx.astype(jnp.int32)
    nidx_i = n_idx.astype(jnp.int32)
    idx_pad = jnp.zeros((NQ_PAD,), jnp.int32).at[:N_QUERY].set(idx_i)
    n_rows = (n_nodes + 127) // 128
    nlu_packed = jnp.zeros((n_rows * 128,), f32).at[:n_nodes].set(
        n_last_update).reshape(n_rows, 128)
    bounds = jnp.searchsorted(
        nidx_i, jnp.arange(0, QPW * (NW + 1), QPW, dtype=jnp.int32)
    ).astype(jnp.int32)
    bounds = jnp.zeros((128,), jnp.int32).at[:NW + 1].set(bounds)

    g, nmem_g, nlu_g = _sc_call(
        table, cidx_i, nidx_i, memb_vals, idx_pad, n_mem, nlu_packed, bounds)

    eb = NQ_PAD // 8
    out = pl.pallas_call(
        _epi_body,
        grid=(8,),
        in_specs=[
            pl.BlockSpec((eb, TPAD), lambda i: (i, 0)),
            pl.BlockSpec((eb, MEM_DIM), lambda i: (i, 0)),
            pl.BlockSpec((eb, 1), lambda i: (i, 0)),
            pl.BlockSpec((eb, 1), lambda i: (i, 0)),
            pl.BlockSpec((1, TIME_DIM), lambda i: (0, 0)),
            pl.BlockSpec((1, TIME_DIM), lambda i: (0, 0)),
            pl.BlockSpec((160, MEM_DIM), lambda i: (0, 0)),
            pl.BlockSpec((1, MEM_DIM), lambda i: (0, 0)),
            pl.BlockSpec((160, MEM_DIM), lambda i: (0, 0)),
            pl.BlockSpec((1, MEM_DIM), lambda i: (0, 0)),
            pl.BlockSpec((2 * MEM_DIM, MEM_DIM), lambda i: (0, 0)),
            pl.BlockSpec((1, MEM_DIM), lambda i: (0, 0)),
        ],
        out_specs=pl.BlockSpec((eb, MEM_DIM), lambda i: (i, 0)),
        out_shape=jax.ShapeDtypeStruct((NQ_PAD, MEM_DIM), f32),
    )(g, nmem_g, nlu_g.reshape(NQ_PAD, 1),
      jnp.zeros((NQ_PAD,), f32).at[:N_QUERY].set(t).reshape(NQ_PAD, 1),
      time_w, time_b.reshape(1, TIME_DIM), n_proj_w,
      n_proj_b.reshape(1, MEM_DIM), c_proj_w, c_proj_b.reshape(1, MEM_DIM),
      out_w, out_b.reshape(1, MEM_DIM))
    return out[:N_QUERY]


# speedup vs baseline: 15.5579x; 1.6891x over previous
import functools

import jax
import jax.numpy as jnp
from jax import lax
from jax.experimental import pallas as pl
from jax.experimental.pallas import tpu as pltpu
from jax.experimental.pallas import tpu_sc as plsc

N_QUERY = 10000
MEM_DIM = 128
TIME_DIM = 32
TW = 192
TPAD = 256
WG = 208
NW = 32
QPW = 320
NQ_PAD = NW * QPW
K = 64
NODE_CHUNK = 64


def _table_body(c_mem_ref, c_lu_ref, cw_ref, ow_ref, tw_ref, t_ref):
    w1u2 = jnp.dot(cw_ref[:MEM_DIM, :], ow_ref[MEM_DIM:, :],
                   preferred_element_type=jnp.float32)
    t_ref[:, :MEM_DIM] = jnp.dot(c_mem_ref[...], w1u2,
                                 preferred_element_type=jnp.float32)
    ph = c_lu_ref[...] * tw_ref[0, :]
    t_ref[:, MEM_DIM:MEM_DIM + TIME_DIM] = jnp.cos(ph)
    t_ref[:, MEM_DIM + TIME_DIM:TW] = jnp.sin(ph)
    t_ref[:, TW:] = jnp.zeros_like(t_ref[:, TW:])


def _epi_body(g_ref, nm_ref, nlu_ref, t_ref, tw_ref, tb_ref, npw_ref, npb_ref,
              cw_ref, cpb_ref, ow_ref, ob_ref, out_ref):
    u1 = ow_ref[:MEM_DIM, :]
    u2 = ow_ref[MEM_DIM:, :]
    wn1u1 = jnp.dot(npw_ref[:MEM_DIM, :], u1, preferred_element_type=jnp.float32)
    wn2u1 = jnp.dot(npw_ref[MEM_DIM:, :], u1, preferred_element_type=jnp.float32)
    w2u2 = jnp.dot(cw_ref[MEM_DIM:, :], u2, preferred_element_type=jnp.float32)
    t = t_ref[...]
    ph_t = t * tw_ref[0, :] + tb_ref[...]
    a = jnp.cos(ph_t)
    b = jnp.sin(ph_t)
    ntime = jnp.cos((t - nlu_ref[...]) * tw_ref[0, :] + tb_ref[...])
    g2 = g_ref[:, MEM_DIM:MEM_DIM + TIME_DIM]
    g3 = g_ref[:, MEM_DIM + TIME_DIM:TW]
    out = jnp.dot(nm_ref[...], wn1u1, preferred_element_type=jnp.float32)
    out += jnp.dot(ntime, wn2u1, preferred_element_type=jnp.float32)
    out += g_ref[:, :MEM_DIM]
    out += jnp.dot(a * g2 + b * g3, w2u2, preferred_element_type=jnp.float32)
    out += g_ref[:, TW:TW + 1] * jnp.dot(cpb_ref[...], u2,
                                         preferred_element_type=jnp.float32)
    out += jnp.dot(npb_ref[...], u1, preferred_element_type=jnp.float32)
    out += ob_ref[...]
    out_ref[...] = out


def _sc_body(t_hbm, cidx_hbm, nidx_hbm, v_hbm, idx_hbm, nmem_hbm, nlu_hbm,
             bounds_hbm, g_hbm, nmemg_hbm, nlug_hbm,
             staging, rows_a, rows_b, cidx_a, cidx_b, nidx_a, nidx_b, v_a,
             v_b, nrows_b, nlub_b, qidx_b, ridx_b, bounds_b,
             sem_ga, sem_gb, sem_ia, sem_ib):
    wid = lax.axis_index("s") * 2 + lax.axis_index("c")
    qbase = wid * QPW
    zeros16 = jnp.zeros((16,), jnp.float32)

    pltpu.sync_copy(bounds_hbm, bounds_b)
    lo = plsc.load_gather(bounds_b, [jnp.full((16,), wid, jnp.int32)])[0]
    hi = plsc.load_gather(bounds_b, [jnp.full((16,), wid + 1, jnp.int32)])[0]

    def _zero_row(r, _):
        for j in range(WG // 16):
            staging[r, pl.ds(16 * j, 16)] = zeros16
        return 0
    lax.fori_loop(0, QPW, _zero_row, 0)

    def _flush(cur_q, accs, acc_s):
        ql = jnp.maximum(cur_q - qbase, 0)
        for j in range(TW // 16):
            staging[ql, pl.ds(16 * j, 16)] = accs[j]
        staging[ql, pl.ds(TW, 16)] = acc_s

    def _idx_descs(c, ci, ni, vi, sem):
        start = pl.multiple_of(c * K, 32)
        return (pltpu.make_async_copy(cidx_hbm.at[pl.ds(start, K)], ci, sem),
                pltpu.make_async_copy(nidx_hbm.at[pl.ds(start, K)], ni, sem),
                pltpu.make_async_copy(v_hbm.at[pl.ds(start, K)], vi, sem))

    def _issue_idx(c, ci, ni, vi, sem):
        for d in _idx_descs(c, ci, ni, vi, sem):
            d.start()

    def _wait_idx(c, ci, ni, vi, sem):
        for d in _idx_descs(c, ci, ni, vi, sem):
            d.wait()

    def _issue_gather(ci, rows, sem):
        pltpu.make_async_copy(t_hbm.at[ci], rows, sem).start()

    def _process(c, ci, ni, vi, rows, sem, carry):
        pltpu.make_async_copy(t_hbm.at[ci], rows, sem).wait()
        start = c * K
        kstart = jnp.maximum(lo - start, 0)
        kend = jnp.minimum(hi - start, K)

        def _group(grp, carry):
            qv = ni[pl.ds(16 * grp, 16)]
            vv = vi[pl.ds(16 * grp, 16)]
            cur_q = carry[0]
            accs = list(carry[1:1 + TW // 16])
            acc_s = carry[1 + TW // 16]
            for lane in range(16):
                k = 16 * grp + lane
                in_rng = jnp.logical_and(k >= kstart, k < kend)
                q = jnp.where(in_rng, qv[lane], cur_q)
                changed = q != cur_q

                @pl.when(changed)
                def _(cur_q=cur_q, accs=accs, acc_s=acc_s):
                    _flush(cur_q, accs, acc_s)

                accs = [jnp.where(changed, zeros16, a) for a in accs]
                acc_s = jnp.where(changed, zeros16, acc_s)
                vs = jnp.where(in_rng, vv[lane], jnp.float32(0.0))
                vsplat = jnp.broadcast_to(vs, (16,))
                for j in range(TW // 16):
                    accs[j] = accs[j] + vsplat * rows[k, pl.ds(16 * j, 16)]
                acc_s = acc_s + vsplat
                cur_q = q
            return (cur_q, *accs, acc_s)

        return lax.fori_loop(0, K // 16, _group, carry)

    carry = (jnp.int32(-1),) + tuple(zeros16 for _ in range(TW // 16 + 1))
    c0 = lo // K
    c1 = (hi + K - 1) // K

    @pl.when(c0 < c1)
    def _():
        _issue_idx(c0, cidx_a, nidx_a, v_a, sem_ia)
        _wait_idx(c0, cidx_a, nidx_a, v_a, sem_ia)
        _issue_gather(cidx_a, rows_a, sem_ga)

        @pl.when(c0 + 1 < c1)
        def _():
            _issue_idx(c0 + 1, cidx_b, nidx_b, v_b, sem_ib)

    npairs = (c1 - c0 + 1) // 2

    def _pair(p, carry):
        ca = c0 + 2 * p
        cb = ca + 1

        @pl.when(cb < c1)
        def _():
            _wait_idx(cb, cidx_b, nidx_b, v_b, sem_ib)
            _issue_gather(cidx_b, rows_b, sem_gb)

        carry = _process(ca, cidx_a, nidx_a, v_a, rows_a, sem_ga, carry)

        @pl.when(ca + 2 < c1)
        def _():
            _issue_idx(ca + 2, cidx_a, nidx_a, v_a, sem_ia)
            _wait_idx(ca + 2, cidx_a, nidx_a, v_a, sem_ia)
            _issue_gather(cidx_a, rows_a, sem_ga)

        carry = lax.cond(
            cb < c1,
            lambda carry: _process(cb, cidx_b, nidx_b, v_b, rows_b, sem_gb,
                                   carry),
            lambda carry: carry, carry)

        @pl.when(cb + 2 < c1)
        def _():
            _issue_idx(cb + 2, cidx_b, nidx_b, v_b, sem_ib)

        return carry

    carry = lax.fori_loop(0, npairs, _pair, carry)
    cur_q = carry[0]

    @pl.when(cur_q >= 0)
    def _():
        _flush(cur_q, list(carry[1:1 + TW // 16]), carry[1 + TW // 16])

    pltpu.sync_copy(staging, g_hbm.at[pl.ds(qbase, QPW)])

    for nc in range(QPW // NODE_CHUNK):
        nb = qbase + NODE_CHUNK * nc
        pltpu.sync_copy(idx_hbm.at[pl.ds(nb, NODE_CHUNK)], qidx_b)
        pltpu.async_copy(nmem_hbm.at[qidx_b], nrows_b, sem_ga).wait()
        pltpu.sync_copy(nrows_b, nmemg_hbm.at[pl.ds(nb, NODE_CHUNK)])
        for grp in range(NODE_CHUNK // 16):
            qv = qidx_b[pl.ds(16 * grp, 16)]
            ridx_b[pl.ds(16 * grp, 16)] = lax.shift_right_logical(
                qv, jnp.full((16,), 7, jnp.int32))
        pltpu.async_copy(nlu_hbm.at[ridx_b], nrows_b, sem_ga).wait()
        for grp in range(NODE_CHUNK // 16):
            qv = qidx_b[pl.ds(16 * grp, 16)]
            col = lax.bitwise_and(qv, jnp.full((16,), 127, jnp.int32))
            row = lax.broadcasted_iota(jnp.int32, (16,), 0) + 16 * grp
            nlub_b[pl.ds(16 * grp, 16)] = plsc.load_gather(
                nrows_b, [row, col])
        pltpu.sync_copy(nlub_b, nlug_hbm.at[pl.ds(nb, NODE_CHUNK)])


_sc_call = functools.partial(
    pl.kernel,
    out_type=(
        jax.ShapeDtypeStruct((NQ_PAD, WG), jnp.float32),
        jax.ShapeDtypeStruct((NQ_PAD, MEM_DIM), jnp.float32),
        jax.ShapeDtypeStruct((NQ_PAD,), jnp.float32),
    ),
    mesh=plsc.VectorSubcoreMesh(core_axis_name="c", subcore_axis_name="s"),
    compiler_params=pltpu.CompilerParams(needs_layout_passes=False),
    scratch_types=[
        pltpu.VMEM((QPW, WG), jnp.float32),
        pltpu.VMEM((K, TPAD), jnp.float32),
        pltpu.VMEM((K, TPAD), jnp.float32),
        pltpu.VMEM((K,), jnp.int32),
        pltpu.VMEM((K,), jnp.int32),
        pltpu.VMEM((K,), jnp.int32),
        pltpu.VMEM((K,), jnp.int32),
        pltpu.VMEM((K,), jnp.float32),
        pltpu.VMEM((K,), jnp.float32),
        pltpu.VMEM((NODE_CHUNK, MEM_DIM), jnp.float32),
        pltpu.VMEM((NODE_CHUNK,), jnp.float32),
        pltpu.VMEM((NODE_CHUNK,), jnp.int32),
        pltpu.VMEM((NODE_CHUNK,), jnp.int32),
        pltpu.VMEM((128,), jnp.int32),
        pltpu.SemaphoreType.DMA,
        pltpu.SemaphoreType.DMA,
        pltpu.SemaphoreType.DMA,
        pltpu.SemaphoreType.DMA,
    ],
)(_sc_body)


def kernel(n_mem, n_last_update, c_mem, c_last_update, idx, t, n_idx, c_idx,
           memb_vals, time_w, time_b, n_proj_w, n_proj_b, c_proj_w, c_proj_b,
           out_w, out_b):
    n_comm = c_mem.shape[0]
    n_nodes = n_mem.shape[0]
    f32 = jnp.float32

    tb = n_comm // 5
    table = pl.pallas_call(
        _table_body,
        grid=(5,),
        in_specs=[
            pl.BlockSpec((tb, MEM_DIM), lambda i: (i, 0)),
            pl.BlockSpec((tb, 1), lambda i: (i, 0)),
            pl.BlockSpec((160, MEM_DIM), lambda i: (0, 0)),
            pl.BlockSpec((2 * MEM_DIM, MEM_DIM), lambda i: (0, 0)),
            pl.BlockSpec((1, TIME_DIM), lambda i: (0, 0)),
        ],
        out_specs=pl.BlockSpec((tb, TPAD), lambda i: (i, 0)),
        out_shape=jax.ShapeDtypeStruct((n_comm, TPAD), f32),
    )(c_mem, c_last_update.reshape(n_comm, 1).astype(f32), c_proj_w, out_w,
      time_w)

    idx_i = idx.astype(jnp.int32)
    n_memb = c_idx.shape[0]
    n_memb_pad = ((n_memb + K - 1) // K) * K
    cidx_i = jnp.zeros((n_memb_pad,), jnp.int32).at[:n_memb].set(
        c_idx.astype(jnp.int32))
    nidx_i = n_idx.astype(jnp.int32)
    nidx_pad = jnp.full((n_memb_pad,), NQ_PAD, jnp.int32).at[:n_memb].set(
        nidx_i)
    v_pad = jnp.zeros((n_memb_pad,), jnp.float32).at[:n_memb].set(memb_vals)
    idx_pad = jnp.zeros((NQ_PAD,), jnp.int32).at[:N_QUERY].set(idx_i)
    n_rows = (n_nodes + 127) // 128
    nlu_packed = jnp.zeros((n_rows * 128,), f32).at[:n_nodes].set(
        n_last_update).reshape(n_rows, 128)
    bounds = jnp.searchsorted(
        nidx_i, jnp.arange(0, QPW * (NW + 1), QPW, dtype=jnp.int32)
    ).astype(jnp.int32)
    bounds = jnp.zeros((128,), jnp.int32).at[:NW + 1].set(bounds)

    g, nmem_g, nlu_g = _sc_call(
        table, cidx_i, nidx_pad, v_pad, idx_pad, n_mem, nlu_packed, bounds)

    eb = NQ_PAD // 8
    out = pl.pallas_call(
        _epi_body,
        grid=(8,),
        in_specs=[
            pl.BlockSpec((eb, WG), lambda i: (i, 0)),
            pl.BlockSpec((eb, MEM_DIM), lambda i: (i, 0)),
            pl.BlockSpec((eb, 1), lambda i: (i, 0)),
            pl.BlockSpec((eb, 1), lambda i: (i, 0)),
            pl.BlockSpec((1, TIME_DIM), lambda i: (0, 0)),
            pl.BlockSpec((1, TIME_DIM), lambda i: (0, 0)),
            pl.BlockSpec((160, MEM_DIM), lambda i: (0, 0)),
            pl.BlockSpec((1, MEM_DIM), lambda i: (0, 0)),
            pl.BlockSpec((160, MEM_DIM), lambda i: (0, 0)),
            pl.BlockSpec((1, MEM_DIM), lambda i: (0, 0)),
            pl.BlockSpec((2 * MEM_DIM, MEM_DIM), lambda i: (0, 0)),
            pl.BlockSpec((1, MEM_DIM), lambda i: (0, 0)),
        ],
        out_specs=pl.BlockSpec((eb, MEM_DIM), lambda i: (i, 0)),
        out_shape=jax.ShapeDtypeStruct((NQ_PAD, MEM_DIM), f32),
    )(g, nmem_g, nlu_g.reshape(NQ_PAD, 1),
      jnp.zeros((NQ_PAD,), f32).at[:N_QUERY].set(t).reshape(NQ_PAD, 1),
      time_w, time_b.reshape(1, TIME_DIM), n_proj_w,
      n_proj_b.reshape(1, MEM_DIM), c_proj_w, c_proj_b.reshape(1, MEM_DIM),
      out_w, out_b.reshape(1, MEM_DIM))
    return out[:N_QUERY]


# speedup vs baseline: 16.0128x; 1.0292x over previous
import functools

import jax
import jax.numpy as jnp
from jax import lax
from jax.experimental import pallas as pl
from jax.experimental.pallas import tpu as pltpu
from jax.experimental.pallas import tpu_sc as plsc

N_QUERY = 10000
MEM_DIM = 128
TIME_DIM = 32
TW = 192
TPAD = 256
WG = 208
NW = 32
QPW = 320
NQ_PAD = NW * QPW
K = 64
NODE_CHUNK = 64


def _table_body(c_mem_ref, c_lu_ref, cw_ref, ow_ref, tw_ref, t_ref):
    w1u2 = jnp.dot(cw_ref[:MEM_DIM, :], ow_ref[MEM_DIM:, :],
                   preferred_element_type=jnp.float32)
    t_ref[:, :MEM_DIM] = jnp.dot(c_mem_ref[...], w1u2,
                                 preferred_element_type=jnp.float32)
    ph = c_lu_ref[...] * tw_ref[0, :]
    t_ref[:, MEM_DIM:MEM_DIM + TIME_DIM] = jnp.cos(ph)
    t_ref[:, MEM_DIM + TIME_DIM:TW] = jnp.sin(ph)
    t_ref[:, TW:] = jnp.zeros_like(t_ref[:, TW:])


def _epi_body(g_ref, nm_ref, nlu_ref, t_ref, tw_ref, tb_ref, npw_ref, npb_ref,
              cw_ref, cpb_ref, ow_ref, ob_ref, out_ref):
    u1 = ow_ref[:MEM_DIM, :]
    u2 = ow_ref[MEM_DIM:, :]
    wn1u1 = jnp.dot(npw_ref[:MEM_DIM, :], u1, preferred_element_type=jnp.float32)
    wn2u1 = jnp.dot(npw_ref[MEM_DIM:, :], u1, preferred_element_type=jnp.float32)
    w2u2 = jnp.dot(cw_ref[MEM_DIM:, :], u2, preferred_element_type=jnp.float32)
    t = t_ref[...]
    ph_t = t * tw_ref[0, :] + tb_ref[...]
    a = jnp.cos(ph_t)
    b = jnp.sin(ph_t)
    ntime = jnp.cos((t - nlu_ref[...]) * tw_ref[0, :] + tb_ref[...])
    g2 = g_ref[:, MEM_DIM:MEM_DIM + TIME_DIM]
    g3 = g_ref[:, MEM_DIM + TIME_DIM:TW]
    out = jnp.dot(nm_ref[...], wn1u1, preferred_element_type=jnp.float32)
    out += jnp.dot(ntime, wn2u1, preferred_element_type=jnp.float32)
    out += g_ref[:, :MEM_DIM]
    out += jnp.dot(a * g2 + b * g3, w2u2, preferred_element_type=jnp.float32)
    out += g_ref[:, TW:TW + 1] * jnp.dot(cpb_ref[...], u2,
                                         preferred_element_type=jnp.float32)
    out += jnp.dot(npb_ref[...], u1, preferred_element_type=jnp.float32)
    out += ob_ref[...]
    out_ref[...] = out


def _sc_body(t_hbm, cidx_hbm, nidx_hbm, v_hbm, idx_hbm, nmem_hbm, nlu_hbm,
             bounds_hbm, g_hbm, nmemg_hbm, nlug_hbm,
             staging, rows_a, rows_b, cidx_a, cidx_b, nidx_a, nidx_b, v_a,
             v_b, nrows_b, nlub_b, qidx_b, ridx_b, bounds_b,
             sem_ga, sem_gb, sem_ia, sem_ib):
    wid = lax.axis_index("s") * 2 + lax.axis_index("c")
    qbase = wid * QPW
    zeros16 = jnp.zeros((16,), jnp.float32)

    pltpu.sync_copy(bounds_hbm, bounds_b)
    lo = plsc.load_gather(bounds_b, [jnp.full((16,), wid, jnp.int32)])[0]
    hi = plsc.load_gather(bounds_b, [jnp.full((16,), wid + 1, jnp.int32)])[0]

    def _zero_row(r, _):
        for j in range(WG // 16):
            staging[r, pl.ds(16 * j, 16)] = zeros16
        return 0
    lax.fori_loop(0, QPW, _zero_row, 0)

    def _flush(cur_q, accs, acc_s):
        ql = jnp.maximum(cur_q - qbase, 0)
        for j in range(TW // 16):
            staging[ql, pl.ds(16 * j, 16)] = accs[j]
        staging[ql, pl.ds(TW, 16)] = acc_s

    def _idx_descs(p, ci, ni, vi, sem):
        start = pl.multiple_of(p * 2 * K, 32)
        return (
            pltpu.make_async_copy(cidx_hbm.at[pl.ds(start, 2 * K)], ci, sem),
            pltpu.make_async_copy(nidx_hbm.at[pl.ds(start, 2 * K)], ni, sem),
            pltpu.make_async_copy(v_hbm.at[pl.ds(start, 2 * K)], vi, sem))

    def _issue_idx(p, ci, ni, vi, sem):
        for d in _idx_descs(p, ci, ni, vi, sem):
            d.start()

    def _wait_idx(p, ci, ni, vi, sem):
        for d in _idx_descs(p, ci, ni, vi, sem):
            d.wait()

    def _issue_gather(ci, off, rows, sem):
        pltpu.make_async_copy(t_hbm.at[ci.at[pl.ds(off, K)]], rows, sem
                              ).start()

    def _process(c, ci, off, ni, vi, rows, sem, carry):
        pltpu.make_async_copy(t_hbm.at[ci.at[pl.ds(off, K)]], rows, sem
                              ).wait()
        start = c * K
        kstart = jnp.maximum(lo - start, 0)
        kend = jnp.minimum(hi - start, K)
        g_lo = jnp.clip(kstart // 16, 0, K // 16)
        g_hi = jnp.clip((kend + 15) // 16, g_lo, K // 16)

        def _group(grp, carry):
            qv = ni[pl.ds(off + 16 * grp, 16)]
            vv = vi[pl.ds(off + 16 * grp, 16)]
            cur_q = carry[0]
            accs = list(carry[1:1 + TW // 16])
            acc_s = carry[1 + TW // 16]
            for lane in range(16):
                k = 16 * grp + lane
                in_rng = jnp.logical_and(k >= kstart, k < kend)
                q = jnp.where(in_rng, qv[lane], cur_q)
                changed = q != cur_q

                def _on_change(cur_q=cur_q, accs=accs, acc_s=acc_s):
                    _flush(cur_q, accs, acc_s)
                    return tuple(zeros16 for _ in range(TW // 16 + 1))

                def _keep(accs=accs, acc_s=acc_s):
                    return (*accs, acc_s)

                res = lax.cond(changed, _on_change, _keep)
                accs = list(res[:TW // 16])
                acc_s = res[TW // 16]
                vs = jnp.where(in_rng, vv[lane], jnp.float32(0.0))
                vsplat = jnp.broadcast_to(vs, (16,))
                for j in range(TW // 16):
                    accs[j] = accs[j] + vsplat * rows[k, pl.ds(16 * j, 16)]
                acc_s = acc_s + vsplat
                cur_q = q
            return (cur_q, *accs, acc_s)

        return lax.fori_loop(g_lo, g_hi, _group, carry)

    carry = (jnp.int32(-1),) + tuple(zeros16 for _ in range(TW // 16 + 1))
    c0 = lo // K
    c1 = (hi + K - 1) // K
    p0 = c0 // 2
    p1 = (c1 + 1) // 2

    def _idx_set(p_rel):
        return (cidx_a, nidx_a, v_a) if p_rel % 2 == 0 else (cidx_b, nidx_b,
                                                             v_b)

    @pl.when(p0 < p1)
    def _():
        _issue_idx(p0, cidx_a, nidx_a, v_a, sem_ia)
        _wait_idx(p0, cidx_a, nidx_a, v_a, sem_ia)
        _issue_gather(cidx_a, 0, rows_a, sem_ga)

    def _pair2(u, carry):
        for par in range(2):
            p = 2 * u + p0 + par
            ci, ni, vi = _idx_set(par)
            oci, oni, ovi = _idx_set(par + 1)
            osem = sem_ib if par == 0 else sem_ia
            ca = 2 * p
            cb = ca + 1
            in_p = p < p1

            @pl.when(jnp.logical_and(in_p, cb < c1))
            def _(ci=ci):
                _issue_gather(ci, K, rows_b, sem_gb)

            @pl.when(p + 1 < p1)
            def _(oci=oci, oni=oni, ovi=ovi, osem=osem, p=p):
                _issue_idx(p + 1, oci, oni, ovi, osem)

            carry = lax.cond(
                in_p,
                lambda carry, p=p, ca=ca, ci=ci, ni=ni, vi=vi:
                    _process(ca, ci, 0, ni, vi, rows_a, sem_ga, carry),
                lambda carry: carry, carry)

            @pl.when(p + 1 < p1)
            def _(oci=oci, oni=oni, ovi=ovi, osem=osem, p=p):
                _wait_idx(p + 1, oci, oni, ovi, osem)
                _issue_gather(oci, 0, rows_a, sem_ga)

            carry = lax.cond(
                jnp.logical_and(in_p, cb < c1),
                lambda carry, cb=cb, ci=ci, ni=ni, vi=vi:
                    _process(cb, ci, K, ni, vi, rows_b, sem_gb, carry),
                lambda carry: carry, carry)
        return carry

    nsteps = (p1 - p0 + 1) // 2
    carry = lax.fori_loop(0, nsteps, _pair2, carry)
    cur_q = carry[0]

    @pl.when(cur_q >= 0)
    def _():
        _flush(cur_q, list(carry[1:1 + TW // 16]), carry[1 + TW // 16])

    pltpu.sync_copy(staging, g_hbm.at[pl.ds(qbase, QPW)])

    for nc in range(QPW // NODE_CHUNK):
        nb = qbase + NODE_CHUNK * nc
        pltpu.sync_copy(idx_hbm.at[pl.ds(nb, NODE_CHUNK)], qidx_b)
        pltpu.async_copy(nmem_hbm.at[qidx_b], nrows_b, sem_ga).wait()
        pltpu.sync_copy(nrows_b, nmemg_hbm.at[pl.ds(nb, NODE_CHUNK)])
        for grp in range(NODE_CHUNK // 16):
            qv = qidx_b[pl.ds(16 * grp, 16)]
            ridx_b[pl.ds(16 * grp, 16)] = lax.shift_right_logical(
                qv, jnp.full((16,), 7, jnp.int32))
        pltpu.async_copy(nlu_hbm.at[ridx_b], nrows_b, sem_ga).wait()
        for grp in range(NODE_CHUNK // 16):
            qv = qidx_b[pl.ds(16 * grp, 16)]
            col = lax.bitwise_and(qv, jnp.full((16,), 127, jnp.int32))
            row = lax.broadcasted_iota(jnp.int32, (16,), 0) + 16 * grp
            nlub_b[pl.ds(16 * grp, 16)] = plsc.load_gather(
                nrows_b, [row, col])
        pltpu.sync_copy(nlub_b, nlug_hbm.at[pl.ds(nb, NODE_CHUNK)])


_sc_call = functools.partial(
    pl.kernel,
    out_type=(
        jax.ShapeDtypeStruct((NQ_PAD, WG), jnp.float32),
        jax.ShapeDtypeStruct((NQ_PAD, MEM_DIM), jnp.float32),
        jax.ShapeDtypeStruct((NQ_PAD,), jnp.float32),
    ),
    mesh=plsc.VectorSubcoreMesh(core_axis_name="c", subcore_axis_name="s"),
    compiler_params=pltpu.CompilerParams(needs_layout_passes=False),
    scratch_types=[
        pltpu.VMEM((QPW, WG), jnp.float32),
        pltpu.VMEM((K, TPAD), jnp.float32),
        pltpu.VMEM((K, TPAD), jnp.float32),
        pltpu.VMEM((2 * K,), jnp.int32),
        pltpu.VMEM((2 * K,), jnp.int32),
        pltpu.VMEM((2 * K,), jnp.int32),
        pltpu.VMEM((2 * K,), jnp.int32),
        pltpu.VMEM((2 * K,), jnp.float32),
        pltpu.VMEM((2 * K,), jnp.float32),
        pltpu.VMEM((NODE_CHUNK, MEM_DIM), jnp.float32),
        pltpu.VMEM((NODE_CHUNK,), jnp.float32),
        pltpu.VMEM((NODE_CHUNK,), jnp.int32),
        pltpu.VMEM((NODE_CHUNK,), jnp.int32),
        pltpu.VMEM((128,), jnp.int32),
        pltpu.SemaphoreType.DMA,
        pltpu.SemaphoreType.DMA,
        pltpu.SemaphoreType.DMA,
        pltpu.SemaphoreType.DMA,
    ],
)(_sc_body)


def kernel(n_mem, n_last_update, c_mem, c_last_update, idx, t, n_idx, c_idx,
           memb_vals, time_w, time_b, n_proj_w, n_proj_b, c_proj_w, c_proj_b,
           out_w, out_b):
    n_comm = c_mem.shape[0]
    n_nodes = n_mem.shape[0]
    f32 = jnp.float32

    tb = n_comm // 5
    table = pl.pallas_call(
        _table_body,
        grid=(5,),
        in_specs=[
            pl.BlockSpec((tb, MEM_DIM), lambda i: (i, 0)),
            pl.BlockSpec((tb, 1), lambda i: (i, 0)),
            pl.BlockSpec((160, MEM_DIM), lambda i: (0, 0)),
            pl.BlockSpec((2 * MEM_DIM, MEM_DIM), lambda i: (0, 0)),
            pl.BlockSpec((1, TIME_DIM), lambda i: (0, 0)),
        ],
        out_specs=pl.BlockSpec((tb, TPAD), lambda i: (i, 0)),
        out_shape=jax.ShapeDtypeStruct((n_comm, TPAD), f32),
    )(c_mem, c_last_update.reshape(n_comm, 1).astype(f32), c_proj_w, out_w,
      time_w)

    idx_i = idx.astype(jnp.int32)
    n_memb = c_idx.shape[0]
    n_memb_pad = ((n_memb + 2 * K - 1) // (2 * K)) * (2 * K)
    cidx_i = jnp.zeros((n_memb_pad,), jnp.int32).at[:n_memb].set(
        c_idx.astype(jnp.int32))
    nidx_i = n_idx.astype(jnp.int32)
    nidx_pad = jnp.full((n_memb_pad,), NQ_PAD, jnp.int32).at[:n_memb].set(
        nidx_i)
    v_pad = jnp.zeros((n_memb_pad,), jnp.float32).at[:n_memb].set(memb_vals)
    idx_pad = jnp.zeros((NQ_PAD,), jnp.int32).at[:N_QUERY].set(idx_i)
    n_rows = (n_nodes + 127) // 128
    nlu_packed = jnp.zeros((n_rows * 128,), f32).at[:n_nodes].set(
        n_last_update).reshape(n_rows, 128)
    bounds = jnp.searchsorted(
        nidx_i, jnp.arange(0, QPW * (NW + 1), QPW, dtype=jnp.int32)
    ).astype(jnp.int32)
    bounds = jnp.zeros((128,), jnp.int32).at[:NW + 1].set(bounds)

    g, nmem_g, nlu_g = _sc_call(
        table, cidx_i, nidx_pad, v_pad, idx_pad, n_mem, nlu_packed, bounds)

    eb = NQ_PAD // 8
    out = pl.pallas_call(
        _epi_body,
        grid=(8,),
        in_specs=[
            pl.BlockSpec((eb, WG), lambda i: (i, 0)),
            pl.BlockSpec((eb, MEM_DIM), lambda i: (i, 0)),
            pl.BlockSpec((eb, 1), lambda i: (i, 0)),
            pl.BlockSpec((eb, 1), lambda i: (i, 0)),
            pl.BlockSpec((1, TIME_DIM), lambda i: (0, 0)),
            pl.BlockSpec((1, TIME_DIM), lambda i: (0, 0)),
            pl.BlockSpec((160, MEM_DIM), lambda i: (0, 0)),
            pl.BlockSpec((1, MEM_DIM), lambda i: (0, 0)),
            pl.BlockSpec((160, MEM_DIM), lambda i: (0, 0)),
            pl.BlockSpec((1, MEM_DIM), lambda i: (0, 0)),
            pl.BlockSpec((2 * MEM_DIM, MEM_DIM), lambda i: (0, 0)),
            pl.BlockSpec((1, MEM_DIM), lambda i: (0, 0)),
        ],
        out_specs=pl.BlockSpec((eb, MEM_DIM), lambda i: (i, 0)),
        out_shape=jax.ShapeDtypeStruct((NQ_PAD, MEM_DIM), f32),
    )(g, nmem_g, nlu_g.reshape(NQ_PAD, 1),
      jnp.zeros((NQ_PAD,), f32).at[:N_QUERY].set(t).reshape(NQ_PAD, 1),
      time_w, time_b.reshape(1, TIME_DIM), n_proj_w,
      n_proj_b.reshape(1, MEM_DIM), c_proj_w, c_proj_b.reshape(1, MEM_DIM),
      out_w, out_b.reshape(1, MEM_DIM))
    return out[:N_QUERY]


# speedup vs baseline: 16.0830x; 1.0044x over previous
import functools

import jax
import jax.numpy as jnp
from jax import lax
from jax.experimental import pallas as pl
from jax.experimental.pallas import tpu as pltpu
from jax.experimental.pallas import tpu_sc as plsc

N_QUERY = 10000
MEM_DIM = 128
TIME_DIM = 32
TW = 192
TPAD = 256
WG = 208
NW = 32
QPW = 320
NQ_PAD = NW * QPW
K = 64
NODE_CHUNK = 64


def _table_body(c_mem_ref, c_lu_ref, cw_ref, ow_ref, tw_ref, t_ref):
    w1u2 = jnp.dot(cw_ref[:MEM_DIM, :], ow_ref[MEM_DIM:, :],
                   preferred_element_type=jnp.float32)
    t_ref[:, :MEM_DIM] = jnp.dot(c_mem_ref[...], w1u2,
                                 preferred_element_type=jnp.float32)
    ph = c_lu_ref[...] * tw_ref[0, :]
    t_ref[:, MEM_DIM:MEM_DIM + TIME_DIM] = jnp.cos(ph)
    t_ref[:, MEM_DIM + TIME_DIM:TW] = jnp.sin(ph)
    t_ref[:, TW:] = jnp.zeros_like(t_ref[:, TW:])


def _epi_body(g_ref, nm_ref, nlu_ref, t_ref, tw_ref, tb_ref, npw_ref, npb_ref,
              cw_ref, cpb_ref, ow_ref, ob_ref, out_ref):
    u1 = ow_ref[:MEM_DIM, :]
    u2 = ow_ref[MEM_DIM:, :]
    wn1u1 = jnp.dot(npw_ref[:MEM_DIM, :], u1, preferred_element_type=jnp.float32)
    wn2u1 = jnp.dot(npw_ref[MEM_DIM:, :], u1, preferred_element_type=jnp.float32)
    w2u2 = jnp.dot(cw_ref[MEM_DIM:, :], u2, preferred_element_type=jnp.float32)
    t = t_ref[...]
    ph_t = t * tw_ref[0, :] + tb_ref[...]
    a = jnp.cos(ph_t)
    b = jnp.sin(ph_t)
    ntime = jnp.cos((t - nlu_ref[...]) * tw_ref[0, :] + tb_ref[...])
    g2 = g_ref[:, MEM_DIM:MEM_DIM + TIME_DIM]
    g3 = g_ref[:, MEM_DIM + TIME_DIM:TW]
    out = jnp.dot(nm_ref[...], wn1u1, preferred_element_type=jnp.float32)
    out += jnp.dot(ntime, wn2u1, preferred_element_type=jnp.float32)
    out += g_ref[:, :MEM_DIM]
    out += jnp.dot(a * g2 + b * g3, w2u2, preferred_element_type=jnp.float32)
    out += g_ref[:, TW:TW + 1] * jnp.dot(cpb_ref[...], u2,
                                         preferred_element_type=jnp.float32)
    out += jnp.dot(npb_ref[...], u1, preferred_element_type=jnp.float32)
    out += ob_ref[...]
    out_ref[...] = out


def _sc_body(t_hbm, cidx_hbm, nidx_hbm, v_hbm, idx_hbm, nmem_hbm, nlu_hbm,
             bounds_hbm, g_hbm, nmemg_hbm, nlug_hbm,
             staging, rows_a, rows_b, cidx_a, cidx_b, nidx_a, nidx_b, v_a,
             v_b, nrows_b, nlub_b, qidx_b, ridx_b, bounds_b,
             sem_ga, sem_gb, sem_ia, sem_ib):
    wid = lax.axis_index("s") * 2 + lax.axis_index("c")
    qbase = wid * QPW
    zeros16 = jnp.zeros((16,), jnp.float32)

    pltpu.sync_copy(bounds_hbm, bounds_b)
    lo = plsc.load_gather(bounds_b, [jnp.full((16,), wid, jnp.int32)])[0]
    hi = plsc.load_gather(bounds_b, [jnp.full((16,), wid + 1, jnp.int32)])[0]

    def _zero_row(r, _):
        for j in range(WG // 16):
            staging[r, pl.ds(16 * j, 16)] = zeros16
        return 0
    lax.fori_loop(0, QPW, _zero_row, 0)

    def _flush(cur_q, accs, acc_s):
        ql = jnp.where(cur_q >= 0, cur_q - qbase, QPW)
        for j in range(TW // 16):
            staging[ql, pl.ds(16 * j, 16)] = accs[j]
        staging[ql, pl.ds(TW, 16)] = acc_s

    def _idx_descs(p, ci, ni, vi, sem):
        start = pl.multiple_of(p * 2 * K, 32)
        return (
            pltpu.make_async_copy(cidx_hbm.at[pl.ds(start, 2 * K)], ci, sem),
            pltpu.make_async_copy(nidx_hbm.at[pl.ds(start, 2 * K)], ni, sem),
            pltpu.make_async_copy(v_hbm.at[pl.ds(start, 2 * K)], vi, sem))

    def _issue_idx(p, ci, ni, vi, sem):
        for d in _idx_descs(p, ci, ni, vi, sem):
            d.start()

    def _wait_idx(p, ci, ni, vi, sem):
        for d in _idx_descs(p, ci, ni, vi, sem):
            d.wait()

    def _issue_gather(ci, off, rows, sem):
        pltpu.make_async_copy(t_hbm.at[ci.at[pl.ds(off, K)]], rows, sem
                              ).start()

    def _process(c, ci, off, ni, vi, rows, sem, carry):
        pltpu.make_async_copy(t_hbm.at[ci.at[pl.ds(off, K)]], rows, sem
                              ).wait()
        start = c * K
        kstart = jnp.maximum(lo - start, 0)
        kend = jnp.minimum(hi - start, K)
        g_lo = jnp.clip(kstart // 16, 0, K // 16)
        g_hi = jnp.clip((kend + 15) // 16, g_lo, K // 16)

        def _group(grp, carry):
            qv = ni[pl.ds(off + 16 * grp, 16)]
            vv = vi[pl.ds(off + 16 * grp, 16)]
            cur_q = carry[0]
            accs = list(carry[1:1 + TW // 16])
            acc_s = carry[1 + TW // 16]
            for lane in range(16):
                k = 16 * grp + lane
                in_rng = jnp.logical_and(k >= kstart, k < kend)
                q = jnp.where(in_rng, qv[lane], cur_q)
                changed = q != cur_q

                def _on_change(cur_q=cur_q, accs=accs, acc_s=acc_s):
                    _flush(cur_q, accs, acc_s)
                    return tuple(zeros16 for _ in range(TW // 16 + 1))

                def _keep(accs=accs, acc_s=acc_s):
                    return (*accs, acc_s)

                res = lax.cond(changed, _on_change, _keep)
                accs = list(res[:TW // 16])
                acc_s = res[TW // 16]
                vs = jnp.where(in_rng, vv[lane], jnp.float32(0.0))
                vsplat = jnp.broadcast_to(vs, (16,))
                for j in range(TW // 16):
                    accs[j] = accs[j] + vsplat * rows[k, pl.ds(16 * j, 16)]
                acc_s = acc_s + vsplat
                cur_q = q
            return (cur_q, *accs, acc_s)

        return plsc.parallel_loop(g_lo, g_hi, carry=carry)(_group)

    carry = (jnp.int32(-1),) + tuple(zeros16 for _ in range(TW // 16 + 1))
    c0 = lo // K
    c1 = (hi + K - 1) // K
    p0 = c0 // 2
    p1 = (c1 + 1) // 2

    def _idx_set(p_rel):
        return (cidx_a, nidx_a, v_a) if p_rel % 2 == 0 else (cidx_b, nidx_b,
                                                             v_b)

    @pl.when(p0 < p1)
    def _():
        _issue_idx(p0, cidx_a, nidx_a, v_a, sem_ia)
        _wait_idx(p0, cidx_a, nidx_a, v_a, sem_ia)
        _issue_gather(cidx_a, 0, rows_a, sem_ga)

    def _pair2(u, carry):
        for par in range(2):
            p = 2 * u + p0 + par
            ci, ni, vi = _idx_set(par)
            oci, oni, ovi = _idx_set(par + 1)
            osem = sem_ib if par == 0 else sem_ia
            ca = 2 * p
            cb = ca + 1
            in_p = p < p1

            @pl.when(jnp.logical_and(in_p, cb < c1))
            def _(ci=ci):
                _issue_gather(ci, K, rows_b, sem_gb)

            @pl.when(p + 1 < p1)
            def _(oci=oci, oni=oni, ovi=ovi, osem=osem, p=p):
                _issue_idx(p + 1, oci, oni, ovi, osem)

            carry = lax.cond(
                in_p,
                lambda carry, p=p, ca=ca, ci=ci, ni=ni, vi=vi:
                    _process(ca, ci, 0, ni, vi, rows_a, sem_ga, carry),
                lambda carry: carry, carry)

            @pl.when(p + 1 < p1)
            def _(oci=oci, oni=oni, ovi=ovi, osem=osem, p=p):
                _wait_idx(p + 1, oci, oni, ovi, osem)
                _issue_gather(oci, 0, rows_a, sem_ga)

            carry = lax.cond(
                jnp.logical_and(in_p, cb < c1),
                lambda carry, cb=cb, ci=ci, ni=ni, vi=vi:
                    _process(cb, ci, K, ni, vi, rows_b, sem_gb, carry),
                lambda carry: carry, carry)
        return carry

    nsteps = (p1 - p0 + 1) // 2
    carry = lax.fori_loop(0, nsteps, _pair2, carry)
    cur_q = carry[0]

    @pl.when(cur_q >= 0)
    def _():
        _flush(cur_q, list(carry[1:1 + TW // 16]), carry[1 + TW // 16])

    pltpu.sync_copy(staging.at[pl.ds(0, QPW)], g_hbm.at[pl.ds(qbase, QPW)])

    for nc in range(QPW // NODE_CHUNK):
        nb = qbase + NODE_CHUNK * nc
        pltpu.sync_copy(idx_hbm.at[pl.ds(nb, NODE_CHUNK)], qidx_b)
        pltpu.async_copy(nmem_hbm.at[qidx_b], nrows_b, sem_ga).wait()
        pltpu.sync_copy(nrows_b, nmemg_hbm.at[pl.ds(nb, NODE_CHUNK)])
        for grp in range(NODE_CHUNK // 16):
            qv = qidx_b[pl.ds(16 * grp, 16)]
            ridx_b[pl.ds(16 * grp, 16)] = lax.shift_right_logical(
                qv, jnp.full((16,), 7, jnp.int32))
        pltpu.async_copy(nlu_hbm.at[ridx_b], nrows_b, sem_ga).wait()
        for grp in range(NODE_CHUNK // 16):
            qv = qidx_b[pl.ds(16 * grp, 16)]
            col = lax.bitwise_and(qv, jnp.full((16,), 127, jnp.int32))
            row = lax.broadcasted_iota(jnp.int32, (16,), 0) + 16 * grp
            nlub_b[pl.ds(16 * grp, 16)] = plsc.load_gather(
                nrows_b, [row, col])
        pltpu.sync_copy(nlub_b, nlug_hbm.at[pl.ds(nb, NODE_CHUNK)])


_sc_call = functools.partial(
    pl.kernel,
    out_type=(
        jax.ShapeDtypeStruct((NQ_PAD, WG), jnp.float32),
        jax.ShapeDtypeStruct((NQ_PAD, MEM_DIM), jnp.float32),
        jax.ShapeDtypeStruct((NQ_PAD,), jnp.float32),
    ),
    mesh=plsc.VectorSubcoreMesh(core_axis_name="c", subcore_axis_name="s"),
    compiler_params=pltpu.CompilerParams(needs_layout_passes=False),
    scratch_types=[
        pltpu.VMEM((QPW + 8, WG), jnp.float32),
        pltpu.VMEM((K, TPAD), jnp.float32),
        pltpu.VMEM((K, TPAD), jnp.float32),
        pltpu.VMEM((2 * K,), jnp.int32),
        pltpu.VMEM((2 * K,), jnp.int32),
        pltpu.VMEM((2 * K,), jnp.int32),
        pltpu.VMEM((2 * K,), jnp.int32),
        pltpu.VMEM((2 * K,), jnp.float32),
        pltpu.VMEM((2 * K,), jnp.float32),
        pltpu.VMEM((NODE_CHUNK, MEM_DIM), jnp.float32),
        pltpu.VMEM((NODE_CHUNK,), jnp.float32),
        pltpu.VMEM((NODE_CHUNK,), jnp.int32),
        pltpu.VMEM((NODE_CHUNK,), jnp.int32),
        pltpu.VMEM((128,), jnp.int32),
        pltpu.SemaphoreType.DMA,
        pltpu.SemaphoreType.DMA,
        pltpu.SemaphoreType.DMA,
        pltpu.SemaphoreType.DMA,
    ],
)(_sc_body)


def kernel(n_mem, n_last_update, c_mem, c_last_update, idx, t, n_idx, c_idx,
           memb_vals, time_w, time_b, n_proj_w, n_proj_b, c_proj_w, c_proj_b,
           out_w, out_b):
    n_comm = c_mem.shape[0]
    n_nodes = n_mem.shape[0]
    f32 = jnp.float32

    tb = n_comm // 5
    table = pl.pallas_call(
        _table_body,
        grid=(5,),
        in_specs=[
            pl.BlockSpec((tb, MEM_DIM), lambda i: (i, 0)),
            pl.BlockSpec((tb, 1), lambda i: (i, 0)),
            pl.BlockSpec((160, MEM_DIM), lambda i: (0, 0)),
            pl.BlockSpec((2 * MEM_DIM, MEM_DIM), lambda i: (0, 0)),
            pl.BlockSpec((1, TIME_DIM), lambda i: (0, 0)),
        ],
        out_specs=pl.BlockSpec((tb, TPAD), lambda i: (i, 0)),
        out_shape=jax.ShapeDtypeStruct((n_comm, TPAD), f32),
    )(c_mem, c_last_update.reshape(n_comm, 1).astype(f32), c_proj_w, out_w,
      time_w)

    idx_i = idx.astype(jnp.int32)
    n_memb = c_idx.shape[0]
    n_memb_pad = ((n_memb + 2 * K - 1) // (2 * K)) * (2 * K)
    cidx_i = jnp.zeros((n_memb_pad,), jnp.int32).at[:n_memb].set(
        c_idx.astype(jnp.int32))
    nidx_i = n_idx.astype(jnp.int32)
    nidx_pad = jnp.full((n_memb_pad,), NQ_PAD, jnp.int32).at[:n_memb].set(
        nidx_i)
    v_pad = jnp.zeros((n_memb_pad,), jnp.float32).at[:n_memb].set(memb_vals)
    idx_pad = jnp.zeros((NQ_PAD,), jnp.int32).at[:N_QUERY].set(idx_i)
    n_rows = (n_nodes + 127) // 128
    nlu_packed = jnp.zeros((n_rows * 128,), f32).at[:n_nodes].set(
        n_last_update).reshape(n_rows, 128)
    bounds = jnp.searchsorted(
        nidx_i, jnp.arange(0, QPW * (NW + 1), QPW, dtype=jnp.int32)
    ).astype(jnp.int32)
    bounds = jnp.zeros((128,), jnp.int32).at[:NW + 1].set(bounds)

    g, nmem_g, nlu_g = _sc_call(
        table, cidx_i, nidx_pad, v_pad, idx_pad, n_mem, nlu_packed, bounds)

    eb = NQ_PAD // 8
    out = pl.pallas_call(
        _epi_body,
        grid=(8,),
        in_specs=[
            pl.BlockSpec((eb, WG), lambda i: (i, 0)),
            pl.BlockSpec((eb, MEM_DIM), lambda i: (i, 0)),
            pl.BlockSpec((eb, 1), lambda i: (i, 0)),
            pl.BlockSpec((eb, 1), lambda i: (i, 0)),
            pl.BlockSpec((1, TIME_DIM), lambda i: (0, 0)),
            pl.BlockSpec((1, TIME_DIM), lambda i: (0, 0)),
            pl.BlockSpec((160, MEM_DIM), lambda i: (0, 0)),
            pl.BlockSpec((1, MEM_DIM), lambda i: (0, 0)),
            pl.BlockSpec((160, MEM_DIM), lambda i: (0, 0)),
            pl.BlockSpec((1, MEM_DIM), lambda i: (0, 0)),
            pl.BlockSpec((2 * MEM_DIM, MEM_DIM), lambda i: (0, 0)),
            pl.BlockSpec((1, MEM_DIM), lambda i: (0, 0)),
        ],
        out_specs=pl.BlockSpec((eb, MEM_DIM), lambda i: (i, 0)),
        out_shape=jax.ShapeDtypeStruct((NQ_PAD, MEM_DIM), f32),
    )(g, nmem_g, nlu_g.reshape(NQ_PAD, 1),
      jnp.zeros((NQ_PAD,), f32).at[:N_QUERY].set(t).reshape(NQ_PAD, 1),
      time_w, time_b.reshape(1, TIME_DIM), n_proj_w,
      n_proj_b.reshape(1, MEM_DIM), c_proj_w, c_proj_b.reshape(1, MEM_DIM),
      out_w, out_b.reshape(1, MEM_DIM))
    return out[:N_QUERY]


# speedup vs baseline: 16.5126x; 1.0267x over previous
import functools

import jax
import jax.numpy as jnp
from jax import lax
from jax.experimental import pallas as pl
from jax.experimental.pallas import tpu as pltpu
from jax.experimental.pallas import tpu_sc as plsc

N_QUERY = 10000
MEM_DIM = 128
TIME_DIM = 32
TW = 192
TPAD = 256
WG = 208
NW = 32
QPW = 320
NQ_PAD = NW * QPW
K = 64
NODE_CHUNK = 64


def _table_body(c_mem_ref, c_lu_ref, cw_ref, ow_ref, tw_ref, t_ref):
    w1u2 = jnp.dot(cw_ref[:MEM_DIM, :], ow_ref[MEM_DIM:, :],
                   preferred_element_type=jnp.float32)
    t_ref[:, :MEM_DIM] = jnp.dot(c_mem_ref[...], w1u2,
                                 preferred_element_type=jnp.float32)
    ph = c_lu_ref[...] * tw_ref[0, :]
    t_ref[:, MEM_DIM:MEM_DIM + TIME_DIM] = jnp.cos(ph)
    t_ref[:, MEM_DIM + TIME_DIM:TW] = jnp.sin(ph)
    t_ref[:, TW:] = jnp.zeros_like(t_ref[:, TW:])


def _epi_body(g_ref, nm_ref, nlu_ref, t_ref, tw_ref, tb_ref, npw_ref, npb_ref,
              cw_ref, cpb_ref, ow_ref, ob_ref, out_ref):
    u1 = ow_ref[:MEM_DIM, :]
    u2 = ow_ref[MEM_DIM:, :]
    wn1u1 = jnp.dot(npw_ref[:MEM_DIM, :], u1, preferred_element_type=jnp.float32)
    wn2u1 = jnp.dot(npw_ref[MEM_DIM:, :], u1, preferred_element_type=jnp.float32)
    w2u2 = jnp.dot(cw_ref[MEM_DIM:, :], u2, preferred_element_type=jnp.float32)
    t = t_ref[...]
    ph_t = t * tw_ref[0, :] + tb_ref[...]
    a = jnp.cos(ph_t)
    b = jnp.sin(ph_t)
    ntime = jnp.cos((t - nlu_ref[...]) * tw_ref[0, :] + tb_ref[...])
    g2 = g_ref[:, MEM_DIM:MEM_DIM + TIME_DIM]
    g3 = g_ref[:, MEM_DIM + TIME_DIM:TW]
    out = jnp.dot(nm_ref[...], wn1u1, preferred_element_type=jnp.float32)
    out += jnp.dot(ntime, wn2u1, preferred_element_type=jnp.float32)
    out += g_ref[:, :MEM_DIM]
    out += jnp.dot(a * g2 + b * g3, w2u2, preferred_element_type=jnp.float32)
    out += g_ref[:, TW:TW + 1] * jnp.dot(cpb_ref[...], u2,
                                         preferred_element_type=jnp.float32)
    out += jnp.dot(npb_ref[...], u1, preferred_element_type=jnp.float32)
    out += ob_ref[...]
    out_ref[...] = out


def _sc_body(t_hbm, cidx_hbm, nidx_hbm, v_hbm, idx_hbm, nmem_hbm, nlu_hbm,
             bounds_hbm, g_hbm, nmemg_hbm, nlug_hbm,
             staging, rows_a, rows_b, cidx_a, cidx_b, nidx_a, nidx_b, v_a,
             v_b, nrows_b, nlub_b, qidx_b, ridx_b, bounds_b,
             sem_ga, sem_gb, sem_ia, sem_ib):
    wid = lax.axis_index("s") * 2 + lax.axis_index("c")
    qbase = wid * QPW
    zeros16 = jnp.zeros((16,), jnp.float32)

    pltpu.sync_copy(bounds_hbm, bounds_b)
    lo = plsc.load_gather(bounds_b, [jnp.full((16,), wid, jnp.int32)])[0]
    hi = plsc.load_gather(bounds_b, [jnp.full((16,), wid + 1, jnp.int32)])[0]

    def _zero_row(r, _):
        for j in range(WG // 16):
            staging[r, pl.ds(16 * j, 16)] = zeros16
        return 0
    lax.fori_loop(0, QPW, _zero_row, 0)

    def _flush(cur_q, accs, acc_s):
        ql = jnp.where(cur_q >= 0, cur_q - qbase, QPW)
        for j in range(TW // 16):
            staging[ql, pl.ds(16 * j, 16)] = accs[j]
        staging[ql, pl.ds(TW, 16)] = acc_s

    def _idx_descs(p, ci, ni, vi, sem):
        start = pl.multiple_of(p * 2 * K, 32)
        return (
            pltpu.make_async_copy(cidx_hbm.at[pl.ds(start, 2 * K)], ci, sem),
            pltpu.make_async_copy(nidx_hbm.at[pl.ds(start, 2 * K)], ni, sem),
            pltpu.make_async_copy(v_hbm.at[pl.ds(start, 2 * K)], vi, sem))

    def _issue_idx(p, ci, ni, vi, sem):
        for d in _idx_descs(p, ci, ni, vi, sem):
            d.start()

    def _wait_idx(p, ci, ni, vi, sem):
        for d in _idx_descs(p, ci, ni, vi, sem):
            d.wait()

    def _issue_gather(ci, off, rows, sem):
        pltpu.make_async_copy(t_hbm.at[ci.at[pl.ds(off, K)]], rows, sem
                              ).start()

    def _process(c, ci, off, ni, vi, rows, sem, carry):
        pltpu.make_async_copy(t_hbm.at[ci.at[pl.ds(off, K)]], rows, sem
                              ).wait()
        start = c * K
        kstart = jnp.maximum(lo - start, 0)
        kend = jnp.minimum(hi - start, K)
        g_lo = jnp.clip(kstart // 16, 0, K // 16)
        g_hi = jnp.clip((kend + 15) // 16, g_lo, K // 16)

        def _group(grp, carry):
            qv = ni[pl.ds(off + 16 * grp, 16)]
            vv = vi[pl.ds(off + 16 * grp, 16)]
            cur_q = carry[0]
            accs = list(carry[1:1 + TW // 16])
            acc_s = carry[1 + TW // 16]
            for lane in range(16):
                k = 16 * grp + lane
                in_rng = jnp.logical_and(k >= kstart, k < kend)
                q = jnp.where(in_rng, qv[lane], cur_q)
                changed = q != cur_q
                vs = jnp.where(in_rng, vv[lane], jnp.float32(0.0))
                vsplat = jnp.broadcast_to(vs, (16,))
                for j in range(TW // 16):
                    accs[j] = (jnp.where(changed, zeros16, accs[j])
                               + vsplat * rows[k, pl.ds(16 * j, 16)])
                acc_s = jnp.where(changed, zeros16, acc_s) + vsplat
                ql = jnp.where(q >= 0, q - qbase, QPW)
                for j in range(TW // 16):
                    staging[ql, pl.ds(16 * j, 16)] = accs[j]
                staging[ql, pl.ds(TW, 16)] = acc_s
                cur_q = q
            return (cur_q, *accs, acc_s)

        return lax.fori_loop(g_lo, g_hi, _group, carry)

    carry = (jnp.int32(-1),) + tuple(zeros16 for _ in range(TW // 16 + 1))
    c0 = lo // K
    c1 = (hi + K - 1) // K
    p0 = c0 // 2
    p1 = (c1 + 1) // 2

    def _idx_set(p_rel):
        return (cidx_a, nidx_a, v_a) if p_rel % 2 == 0 else (cidx_b, nidx_b,
                                                             v_b)

    @pl.when(p0 < p1)
    def _():
        _issue_idx(p0, cidx_a, nidx_a, v_a, sem_ia)
        _wait_idx(p0, cidx_a, nidx_a, v_a, sem_ia)
        _issue_gather(cidx_a, 0, rows_a, sem_ga)

    def _pair2(u, carry):
        for par in range(2):
            p = 2 * u + p0 + par
            ci, ni, vi = _idx_set(par)
            oci, oni, ovi = _idx_set(par + 1)
            osem = sem_ib if par == 0 else sem_ia
            ca = 2 * p
            cb = ca + 1
            in_p = p < p1

            @pl.when(jnp.logical_and(in_p, cb < c1))
            def _(ci=ci):
                _issue_gather(ci, K, rows_b, sem_gb)

            @pl.when(p + 1 < p1)
            def _(oci=oci, oni=oni, ovi=ovi, osem=osem, p=p):
                _issue_idx(p + 1, oci, oni, ovi, osem)

            carry = lax.cond(
                in_p,
                lambda carry, p=p, ca=ca, ci=ci, ni=ni, vi=vi:
                    _process(ca, ci, 0, ni, vi, rows_a, sem_ga, carry),
                lambda carry: carry, carry)

            @pl.when(p + 1 < p1)
            def _(oci=oci, oni=oni, ovi=ovi, osem=osem, p=p):
                _wait_idx(p + 1, oci, oni, ovi, osem)
                _issue_gather(oci, 0, rows_a, sem_ga)

            carry = lax.cond(
                jnp.logical_and(in_p, cb < c1),
                lambda carry, cb=cb, ci=ci, ni=ni, vi=vi:
                    _process(cb, ci, K, ni, vi, rows_b, sem_gb, carry),
                lambda carry: carry, carry)
        return carry

    nsteps = (p1 - p0 + 1) // 2
    carry = lax.fori_loop(0, nsteps, _pair2, carry)

    pltpu.sync_copy(staging.at[pl.ds(0, QPW)], g_hbm.at[pl.ds(qbase, QPW)])

    for nc in range(QPW // NODE_CHUNK):
        nb = qbase + NODE_CHUNK * nc
        pltpu.sync_copy(idx_hbm.at[pl.ds(nb, NODE_CHUNK)], qidx_b)
        pltpu.async_copy(nmem_hbm.at[qidx_b], nrows_b, sem_ga).wait()
        pltpu.sync_copy(nrows_b, nmemg_hbm.at[pl.ds(nb, NODE_CHUNK)])
        for grp in range(NODE_CHUNK // 16):
            qv = qidx_b[pl.ds(16 * grp, 16)]
            ridx_b[pl.ds(16 * grp, 16)] = lax.shift_right_logical(
                qv, jnp.full((16,), 7, jnp.int32))
        pltpu.async_copy(nlu_hbm.at[ridx_b], nrows_b, sem_ga).wait()
        for grp in range(NODE_CHUNK // 16):
            qv = qidx_b[pl.ds(16 * grp, 16)]
            col = lax.bitwise_and(qv, jnp.full((16,), 127, jnp.int32))
            row = lax.broadcasted_iota(jnp.int32, (16,), 0) + 16 * grp
            nlub_b[pl.ds(16 * grp, 16)] = plsc.load_gather(
                nrows_b, [row, col])
        pltpu.sync_copy(nlub_b, nlug_hbm.at[pl.ds(nb, NODE_CHUNK)])


_sc_call = functools.partial(
    pl.kernel,
    out_type=(
        jax.ShapeDtypeStruct((NQ_PAD, WG), jnp.float32),
        jax.ShapeDtypeStruct((NQ_PAD, MEM_DIM), jnp.float32),
        jax.ShapeDtypeStruct((NQ_PAD,), jnp.float32),
    ),
    mesh=plsc.VectorSubcoreMesh(core_axis_name="c", subcore_axis_name="s"),
    compiler_params=pltpu.CompilerParams(needs_layout_passes=False),
    scratch_types=[
        pltpu.VMEM((QPW + 8, WG), jnp.float32),
        pltpu.VMEM((K, TPAD), jnp.float32),
        pltpu.VMEM((K, TPAD), jnp.float32),
        pltpu.VMEM((2 * K,), jnp.int32),
        pltpu.VMEM((2 * K,), jnp.int32),
        pltpu.VMEM((2 * K,), jnp.int32),
        pltpu.VMEM((2 * K,), jnp.int32),
        pltpu.VMEM((2 * K,), jnp.float32),
        pltpu.VMEM((2 * K,), jnp.float32),
        pltpu.VMEM((NODE_CHUNK, MEM_DIM), jnp.float32),
        pltpu.VMEM((NODE_CHUNK,), jnp.float32),
        pltpu.VMEM((NODE_CHUNK,), jnp.int32),
        pltpu.VMEM((NODE_CHUNK,), jnp.int32),
        pltpu.VMEM((128,), jnp.int32),
        pltpu.SemaphoreType.DMA,
        pltpu.SemaphoreType.DMA,
        pltpu.SemaphoreType.DMA,
        pltpu.SemaphoreType.DMA,
    ],
)(_sc_body)


def kernel(n_mem, n_last_update, c_mem, c_last_update, idx, t, n_idx, c_idx,
           memb_vals, time_w, time_b, n_proj_w, n_proj_b, c_proj_w, c_proj_b,
           out_w, out_b):
    n_comm = c_mem.shape[0]
    n_nodes = n_mem.shape[0]
    f32 = jnp.float32

    tb = n_comm // 5
    table = pl.pallas_call(
        _table_body,
        grid=(5,),
        in_specs=[
            pl.BlockSpec((tb, MEM_DIM), lambda i: (i, 0)),
            pl.BlockSpec((tb, 1), lambda i: (i, 0)),
            pl.BlockSpec((160, MEM_DIM), lambda i: (0, 0)),
            pl.BlockSpec((2 * MEM_DIM, MEM_DIM), lambda i: (0, 0)),
            pl.BlockSpec((1, TIME_DIM), lambda i: (0, 0)),
        ],
        out_specs=pl.BlockSpec((tb, TPAD), lambda i: (i, 0)),
        out_shape=jax.ShapeDtypeStruct((n_comm, TPAD), f32),
    )(c_mem, c_last_update.reshape(n_comm, 1).astype(f32), c_proj_w, out_w,
      time_w)

    idx_i = idx.astype(jnp.int32)
    n_memb = c_idx.shape[0]
    n_memb_pad = ((n_memb + 2 * K - 1) // (2 * K)) * (2 * K)
    cidx_i = jnp.zeros((n_memb_pad,), jnp.int32).at[:n_memb].set(
        c_idx.astype(jnp.int32))
    nidx_i = n_idx.astype(jnp.int32)
    nidx_pad = jnp.full((n_memb_pad,), NQ_PAD, jnp.int32).at[:n_memb].set(
        nidx_i)
    v_pad = jnp.zeros((n_memb_pad,), jnp.float32).at[:n_memb].set(memb_vals)
    idx_pad = jnp.zeros((NQ_PAD,), jnp.int32).at[:N_QUERY].set(idx_i)
    n_rows = (n_nodes + 127) // 128
    nlu_packed = jnp.zeros((n_rows * 128,), f32).at[:n_nodes].set(
        n_last_update).reshape(n_rows, 128)
    bounds = jnp.searchsorted(
        nidx_i, jnp.arange(0, QPW * (NW + 1), QPW, dtype=jnp.int32)
    ).astype(jnp.int32)
    bounds = jnp.zeros((128,), jnp.int32).at[:NW + 1].set(bounds)

    g, nmem_g, nlu_g = _sc_call(
        table, cidx_i, nidx_pad, v_pad, idx_pad, n_mem, nlu_packed, bounds)

    eb = NQ_PAD // 8
    out = pl.pallas_call(
        _epi_body,
        grid=(8,),
        in_specs=[
            pl.BlockSpec((eb, WG), lambda i: (i, 0)),
            pl.BlockSpec((eb, MEM_DIM), lambda i: (i, 0)),
            pl.BlockSpec((eb, 1), lambda i: (i, 0)),
            pl.BlockSpec((eb, 1), lambda i: (i, 0)),
            pl.BlockSpec((1, TIME_DIM), lambda i: (0, 0)),
            pl.BlockSpec((1, TIME_DIM), lambda i: (0, 0)),
            pl.BlockSpec((160, MEM_DIM), lambda i: (0, 0)),
            pl.BlockSpec((1, MEM_DIM), lambda i: (0, 0)),
            pl.BlockSpec((160, MEM_DIM), lambda i: (0, 0)),
            pl.BlockSpec((1, MEM_DIM), lambda i: (0, 0)),
            pl.BlockSpec((2 * MEM_DIM, MEM_DIM), lambda i: (0, 0)),
            pl.BlockSpec((1, MEM_DIM), lambda i: (0, 0)),
        ],
        out_specs=pl.BlockSpec((eb, MEM_DIM), lambda i: (i, 0)),
        out_shape=jax.ShapeDtypeStruct((NQ_PAD, MEM_DIM), f32),
    )(g, nmem_g, nlu_g.reshape(NQ_PAD, 1),
      jnp.zeros((NQ_PAD,), f32).at[:N_QUERY].set(t).reshape(NQ_PAD, 1),
      time_w, time_b.reshape(1, TIME_DIM), n_proj_w,
      n_proj_b.reshape(1, MEM_DIM), c_proj_w, c_proj_b.reshape(1, MEM_DIM),
      out_w, out_b.reshape(1, MEM_DIM))
    return out[:N_QUERY]
